# baseline (speedup 1.0000x reference)
import functools
import numpy as np
import jax
import jax.numpy as jnp
from jax import lax
from jax.experimental import pallas as pl
from jax.experimental.pallas import tpu as pltpu
from jax.experimental.pallas import tpu_sc as plsc

_B = 16384
_V = 1000

_BTC = 0
_BSC = _B - _BTC

_R = 2048
_NW = 32
_CH_ROWS = 32
_ROWS_W = _BSC // _NW
_N_CHUNKS = _ROWS_W // _CH_ROWS
_NWIN = 62

_MANT_MASK = jnp.int32(0x007FFFFF)
_ONE_BITS = jnp.int32(0x3F800000)
_LN2 = 0.6931471805599453


def _bits(x):
    return lax.bitcast_convert_type(x, jnp.int32)


def _mant(bits):
    return lax.bitcast_convert_type((bits & _MANT_MASK) | _ONE_BITS, jnp.float32)


_sc_mesh = plsc.VectorSubcoreMesh(core_axis_name="c", subcore_axis_name="s")


@functools.partial(
    pl.kernel,
    mesh=_sc_mesh,
    out_type=(
        jax.ShapeDtypeStruct((_NW, 16), jnp.int32),
        jax.ShapeDtypeStruct((_NW, 16), jnp.float32),
        jax.ShapeDtypeStruct((_NW, 16), jnp.int32),
        jax.ShapeDtypeStruct((_NW, 16), jnp.float32),
    ),
    scratch_types=[
        pltpu.VMEM((_CH_ROWS, _V), jnp.float32),
        pltpu.VMEM((_CH_ROWS, _V), jnp.float32),
        pltpu.VMEM((_ROWS_W,), jnp.int32),
        pltpu.VMEM((16,), jnp.int32),
        pltpu.VMEM((16,), jnp.float32),
        pltpu.VMEM((16,), jnp.int32),
        pltpu.VMEM((16,), jnp.float32),
        pltpu.SemaphoreType.DMA,
        pltpu.SemaphoreType.DMA,
        pltpu.SemaphoreType.DMA,
    ],
)
def _sc_dense(a_hbm, y_hbm, oute_hbm, outp_hbm, outec_hbm, outpc_hbm,
              buf0, buf1, ybuf, ev, pv, ecv, pcv, sem0, sem1, semy):
    c = lax.axis_index("c")
    s = lax.axis_index("s")
    wid = s * 2 + c
    row0 = _BTC + wid * _ROWS_W
    bufs = (buf0, buf1)
    sems = (sem0, sem1)

    cpy = pltpu.async_copy(y_hbm.at[pl.ds(row0, _ROWS_W)], ybuf, semy)

    def fire(j):
        return pltpu.async_copy(
            a_hbm.at[pl.ds(row0 + j * _CH_ROWS, _CH_ROWS)], bufs[j % 2], sems[j % 2]
        )

    lane = lax.broadcasted_iota(jnp.int32, (16,), 0)
    mask8 = lane >= 8
    izero = jnp.zeros((16,), jnp.int32)
    fone = jnp.ones((16,), jnp.float32)

    e0 = izero
    e1 = izero
    p0 = fone
    ec = izero
    pc = fone

    pending = {0: fire(0)}
    cpy.wait()
    for j in range(_N_CHUNKS):
        if j + 1 < _N_CHUNKS:
            pending[j + 1] = fire(j + 1)
        pending.pop(j).wait()
        buf = bufs[j % 2]

        def row_body(r, carry):
            e0, e1, p0 = carry
            pa, pb, pcd, pd = p0, fone, fone, fone
            for w in range(_NWIN):
                bits = _bits(1.0 - buf[r, pl.ds(w * 16, 16)])
                if w % 2 == 0:
                    e0 = e0 + lax.shift_right_logical(bits, 23)
                else:
                    e1 = e1 + lax.shift_right_logical(bits, 23)
                m = _mant(bits)
                if w % 4 == 0:
                    pa = pa * m
                elif w % 4 == 1:
                    pb = pb * m
                elif w % 4 == 2:
                    pcd = pcd * m
                else:
                    pd = pd * m
            tb = _bits(1.0 - buf[r, pl.ds(_V - 16, 16)])
            e0 = e0 + jnp.where(mask8, lax.shift_right_logical(tb, 23), izero)
            pa = pa * jnp.where(mask8, _mant(tb), fone)
            comb = (pa * pb) * (pcd * pd)
            cb = _bits(comb)
            e0 = e0 + lax.shift_right_logical(cb, 23)
            return e0, e1, _mant(cb)

        e0, e1, p0 = lax.fori_loop(0, _CH_ROWS, row_body, (e0, e1, p0))

        def corr_group(g, carry):
            ec, pc = carry
            goff = pl.multiple_of(j * _CH_ROWS + g * 16, 16)
            yg = ybuf[pl.ds(goff, 16)]
            for rr in range(16):
                yv = yg[rr]
                start = pl.multiple_of((yv >> 4) << 4, 16)
                lsel = jnp.full((16,), yv & 15, jnp.int32)
                win = buf[g * 16 + rr, pl.ds(start, 16)]
                msel = lane == lsel
                tv = jnp.where(msel, win, fone)
                uv = jnp.where(msel, 1.0 - win, fone)
                btv = _bits(tv)
                buv = _bits(uv)
                ec = (
                    ec
                    + lax.shift_right_logical(btv, 23)
                    - lax.shift_right_logical(buv, 23)
                )
                pc = pc * (_mant(btv) / _mant(buv))
            return ec, pc

        ec, pc = lax.fori_loop(0, _CH_ROWS // 16, corr_group, (ec, pc))
        pcb = _bits(pc)
        ec = ec + lax.shift_right_logical(pcb, 23)
        pc = _mant(pcb)

    ev[...] = e0 + e1
    pv[...] = p0
    ecv[...] = ec
    pcv[...] = pc
    pltpu.async_copy(ev, oute_hbm.at[wid], sem0).wait()
    pltpu.async_copy(pv, outp_hbm.at[wid], sem0).wait()
    pltpu.async_copy(ecv, outec_hbm.at[wid], sem0).wait()
    pltpu.async_copy(pcv, outpc_hbm.at[wid], sem0).wait()


def _sc_bias():
    cnt = np.zeros((_NW, 16), np.int64)
    cnt[:, :8] = _NWIN * _ROWS_W
    cnt[:, 8:] = (_NWIN + 1) * _ROWS_W
    cnt += _ROWS_W
    return (127 * cnt).astype(np.int32)


_SC_BIAS = _sc_bias()
_EC_BIAS = np.int32(127 * _N_CHUNKS)


def _tc_body(y_ref, a_ref, o_ref):
    a = a_ref[...]
    l1 = jnp.maximum(jnp.log(1.0 - a), -100.0)
    y = y_ref[...]
    col = jax.lax.broadcasted_iota(jnp.int32, (_R, _V), 1)
    t = jnp.sum(jnp.where(col == y, a, 0.0), axis=1, keepdims=True)
    lp_t = jnp.maximum(jnp.log(t), -100.0)
    l1_t = jnp.maximum(jnp.log(1.0 - t), -100.0)
    o_ref[...] = (jnp.sum(l1) + jnp.sum(lp_t - l1_t)).reshape(1, 1, 1)


def _combine_body(tc_ref, e_ref, p_ref, ec_ref, pc_ref, bias_ref, o_ref):
    net_e = jnp.sum(e_ref[...] - bias_ref[...])
    s_dense = net_e.astype(jnp.float32) + jnp.sum(jnp.log2(p_ref[...]))
    net_ec = jnp.sum(ec_ref[...] - _EC_BIAS)
    s_corr = net_ec.astype(jnp.float32) + jnp.sum(jnp.log2(pc_ref[...]))
    total = _LN2 * (s_dense + s_corr) + jnp.sum(tc_ref[...])
    o_ref[...] = (-total / (_B * _V)).reshape(1, 1)


def kernel(truth, prob, all_truth, y_target):
    del truth, prob

    esum, prod, ecorr, pcorr = _sc_dense(all_truth, y_target)

    if _BTC > 0:
        y2 = y_target.reshape(_B, 1)
        grid = _BTC // _R
        tc_partials = pl.pallas_call(
            _tc_body,
            grid=(grid,),
            in_specs=[
                pl.BlockSpec((_R, 1), lambda i: (i, 0)),
                pl.BlockSpec((_R, _V), lambda i: (i, 0)),
            ],
            out_specs=pl.BlockSpec((1, 1, 1), lambda i: (i, 0, 0)),
            out_shape=jax.ShapeDtypeStruct((grid, 1, 1), jnp.float32),
            compiler_params=pltpu.CompilerParams(
                dimension_semantics=("parallel",),
            ),
        )(y2, all_truth)
    else:
        tc_partials = jnp.zeros((1, 1, 1), jnp.float32)

    out = pl.pallas_call(
        _combine_body,
        out_shape=jax.ShapeDtypeStruct((1, 1), jnp.float32),
    )(tc_partials, esum, prod, ecorr, pcorr, jnp.asarray(_SC_BIAS))
    return out[0, 0]


# speedup vs baseline: 1.2081x; 1.2081x over previous
import functools
import numpy as np
import jax
import jax.numpy as jnp
from jax import lax
from jax.experimental import pallas as pl
from jax.experimental.pallas import tpu as pltpu
from jax.experimental.pallas import tpu_sc as plsc

_B = 16384
_V = 1000

_BTC = 10240
_BSC = _B - _BTC

_R = 2048
_NW = 32
_CH_ROWS = 32
_ROWS_W = _BSC // _NW
_N_CHUNKS = _ROWS_W // _CH_ROWS
_NWIN = 62

_MANT_MASK = jnp.int32(0x007FFFFF)
_ONE_BITS = jnp.int32(0x3F800000)
_LN2 = 0.6931471805599453


def _bits(x):
    return lax.bitcast_convert_type(x, jnp.int32)


def _mant(bits):
    return lax.bitcast_convert_type((bits & _MANT_MASK) | _ONE_BITS, jnp.float32)


_sc_mesh = plsc.VectorSubcoreMesh(core_axis_name="c", subcore_axis_name="s")


@functools.partial(
    pl.kernel,
    mesh=_sc_mesh,
    out_type=(
        jax.ShapeDtypeStruct((_NW, 16), jnp.int32),
        jax.ShapeDtypeStruct((_NW, 16), jnp.float32),
        jax.ShapeDtypeStruct((_NW, 16), jnp.int32),
        jax.ShapeDtypeStruct((_NW, 16), jnp.float32),
    ),
    scratch_types=[
        pltpu.VMEM((_CH_ROWS, _V), jnp.float32),
        pltpu.VMEM((_CH_ROWS, _V), jnp.float32),
        pltpu.VMEM((_ROWS_W,), jnp.int32),
        pltpu.VMEM((16,), jnp.int32),
        pltpu.VMEM((16,), jnp.float32),
        pltpu.VMEM((16,), jnp.int32),
        pltpu.VMEM((16,), jnp.float32),
        pltpu.SemaphoreType.DMA,
        pltpu.SemaphoreType.DMA,
        pltpu.SemaphoreType.DMA,
    ],
)
def _sc_dense(a_hbm, y_hbm, oute_hbm, outp_hbm, outec_hbm, outpc_hbm,
              buf0, buf1, ybuf, ev, pv, ecv, pcv, sem0, sem1, semy):
    c = lax.axis_index("c")
    s = lax.axis_index("s")
    wid = s * 2 + c
    row0 = _BTC + wid * _ROWS_W
    bufs = (buf0, buf1)
    sems = (sem0, sem1)

    cpy = pltpu.async_copy(y_hbm.at[pl.ds(row0, _ROWS_W)], ybuf, semy)

    def fire(j):
        return pltpu.async_copy(
            a_hbm.at[pl.ds(row0 + j * _CH_ROWS, _CH_ROWS)], bufs[j % 2], sems[j % 2]
        )

    lane = lax.broadcasted_iota(jnp.int32, (16,), 0)
    mask8 = lane >= 8
    izero = jnp.zeros((16,), jnp.int32)
    fone = jnp.ones((16,), jnp.float32)

    e0 = izero
    e1 = izero
    p0 = fone
    ec = izero
    pc = fone

    pending = {0: fire(0)}
    cpy.wait()
    for j in range(_N_CHUNKS):
        if j + 1 < _N_CHUNKS:
            pending[j + 1] = fire(j + 1)
        pending.pop(j).wait()
        buf = bufs[j % 2]

        def row_body(r, carry):
            e0, e1, p0 = carry
            pa, pb, pcd, pd = p0, fone, fone, fone
            for w in range(_NWIN):
                bits = _bits(1.0 - buf[r, pl.ds(w * 16, 16)])
                if w % 2 == 0:
                    e0 = e0 + lax.shift_right_logical(bits, 23)
                else:
                    e1 = e1 + lax.shift_right_logical(bits, 23)
                m = _mant(bits)
                if w % 4 == 0:
                    pa = pa * m
                elif w % 4 == 1:
                    pb = pb * m
                elif w % 4 == 2:
                    pcd = pcd * m
                else:
                    pd = pd * m
            tb = _bits(1.0 - buf[r, pl.ds(_V - 16, 16)])
            e0 = e0 + jnp.where(mask8, lax.shift_right_logical(tb, 23), izero)
            pa = pa * jnp.where(mask8, _mant(tb), fone)
            comb = (pa * pb) * (pcd * pd)
            cb = _bits(comb)
            e0 = e0 + lax.shift_right_logical(cb, 23)
            return e0, e1, _mant(cb)

        e0, e1, p0 = lax.fori_loop(0, _CH_ROWS, row_body, (e0, e1, p0))

        def corr_group(g, carry):
            ec, pc = carry
            goff = pl.multiple_of(j * _CH_ROWS + g * 16, 16)
            yg = ybuf[pl.ds(goff, 16)]
            for rr in range(16):
                yv = yg[rr]
                start = pl.multiple_of((yv >> 4) << 4, 16)
                lsel = jnp.full((16,), yv & 15, jnp.int32)
                win = buf[g * 16 + rr, pl.ds(start, 16)]
                msel = lane == lsel
                tv = jnp.where(msel, win, fone)
                uv = jnp.where(msel, 1.0 - win, fone)
                btv = _bits(tv)
                buv = _bits(uv)
                ec = (
                    ec
                    + lax.shift_right_logical(btv, 23)
                    - lax.shift_right_logical(buv, 23)
                )
                pc = pc * (_mant(btv) / _mant(buv))
            return ec, pc

        ec, pc = lax.fori_loop(0, _CH_ROWS // 16, corr_group, (ec, pc))
        pcb = _bits(pc)
        ec = ec + lax.shift_right_logical(pcb, 23)
        pc = _mant(pcb)

    ev[...] = e0 + e1
    pv[...] = p0
    ecv[...] = ec
    pcv[...] = pc
    pltpu.async_copy(ev, oute_hbm.at[wid], sem0).wait()
    pltpu.async_copy(pv, outp_hbm.at[wid], sem0).wait()
    pltpu.async_copy(ecv, outec_hbm.at[wid], sem0).wait()
    pltpu.async_copy(pcv, outpc_hbm.at[wid], sem0).wait()


def _sc_bias():
    cnt = np.zeros((_NW, 16), np.int64)
    cnt[:, :8] = _NWIN * _ROWS_W
    cnt[:, 8:] = (_NWIN + 1) * _ROWS_W
    cnt += _ROWS_W
    return (127 * cnt).astype(np.int32)


_SC_BIAS = _sc_bias()
_EC_BIAS = np.int32(127 * _N_CHUNKS)


def _tc_body(y_ref, a_ref, o_ref):
    a = a_ref[...]
    l1 = jnp.maximum(jnp.log(1.0 - a), -100.0)
    y = y_ref[...]
    col = jax.lax.broadcasted_iota(jnp.int32, (_R, _V), 1)
    t = jnp.sum(jnp.where(col == y, a, 0.0), axis=1, keepdims=True)
    lp_t = jnp.maximum(jnp.log(t), -100.0)
    l1_t = jnp.maximum(jnp.log(1.0 - t), -100.0)
    o_ref[...] = (jnp.sum(l1) + jnp.sum(lp_t - l1_t)).reshape(1, 1, 1)


def _combine_body(tc_ref, e_ref, p_ref, ec_ref, pc_ref, bias_ref, o_ref):
    net_e = jnp.sum(e_ref[...] - bias_ref[...])
    s_dense = net_e.astype(jnp.float32) + jnp.sum(jnp.log2(p_ref[...]))
    net_ec = jnp.sum(ec_ref[...] - _EC_BIAS)
    s_corr = net_ec.astype(jnp.float32) + jnp.sum(jnp.log2(pc_ref[...]))
    total = _LN2 * (s_dense + s_corr) + jnp.sum(tc_ref[...])
    o_ref[...] = (-total / (_B * _V)).reshape(1, 1)


def kernel(truth, prob, all_truth, y_target):
    del truth, prob

    esum, prod, ecorr, pcorr = _sc_dense(all_truth, y_target)

    if _BTC > 0:
        y2 = y_target.reshape(_B, 1)
        grid = _BTC // _R
        tc_partials = pl.pallas_call(
            _tc_body,
            grid=(grid,),
            in_specs=[
                pl.BlockSpec((_R, 1), lambda i: (i, 0)),
                pl.BlockSpec((_R, _V), lambda i: (i, 0)),
            ],
            out_specs=pl.BlockSpec((1, 1, 1), lambda i: (i, 0, 0)),
            out_shape=jax.ShapeDtypeStruct((grid, 1, 1), jnp.float32),
            compiler_params=pltpu.CompilerParams(
                dimension_semantics=("parallel",),
            ),
        )(y2, all_truth)
    else:
        tc_partials = jnp.zeros((1, 1, 1), jnp.float32)

    out = pl.pallas_call(
        _combine_body,
        out_shape=jax.ShapeDtypeStruct((1, 1), jnp.float32),
    )(tc_partials, esum, prod, ecorr, pcorr, jnp.asarray(_SC_BIAS))
    return out[0, 0]
